# Initial kernel scaffold; baseline (speedup 1.0000x reference)
import jax
import jax.numpy as jnp
from jax.experimental import pallas as pl


def kernel(xyz, params):
    raise NotImplementedError("write your pallas kernel here")


# speedup vs baseline: 17.3247x; 17.3247x over previous
import functools
import jax
import jax.numpy as jnp
from jax.experimental import pallas as pl

_B, _N, _NPART = 32, 2048, 50


def _fold_bn(layer):
    s = layer['gamma'] / jnp.sqrt(layer['var'] + 1e-5)
    wt = (layer['W'] * s[:, None]).T
    b = (layer['b'] - layer['mean']) * s + layer['beta']
    return wt, b


def _pad_rows(wt, to):
    return jnp.pad(wt, ((0, to - wt.shape[0]), (0, 0)))


def _fps_body(n, npoint, xyzT_ref, out_ref):
    xyz = xyzT_ref[...]
    lane = jax.lax.broadcasted_iota(jnp.int32, (_B, n), 1)

    def body(i, state):
        dist, far = state
        onehot = (lane == far).astype(jnp.float32)
        c = jnp.sum(xyz * onehot[:, None, :], axis=2)
        out_ref[:, pl.ds(i, 1), :] = c[:, None, :]
        d = jnp.sum((xyz - c[:, :, None]) ** 2, axis=1)
        dist = jnp.minimum(dist, d)
        m = jnp.max(dist, axis=1, keepdims=True)
        far = jnp.min(jnp.where(dist == m, lane, n),
                      axis=1, keepdims=True).astype(jnp.int32)
        return dist, far

    dist0 = jnp.full((_B, n), 1e10, dtype=jnp.float32)
    far0 = jnp.zeros((_B, 1), dtype=jnp.int32)
    jax.lax.fori_loop(0, npoint, body, (dist0, far0))


def _fps(xyzT, npoint):
    n = xyzT.shape[2]
    return pl.pallas_call(
        functools.partial(_fps_body, n, npoint),
        out_shape=jax.ShapeDtypeStruct((_B, npoint, 8), jnp.float32),
    )(xyzT)


def _sa_body(n, sc, ns, r2, has_feats, *refs):
    if has_feats:
        (xyzp_ref, xyzT_ref, feats_ref, nxyz_ref,
         w1x_ref, w1f_ref, b1_ref, w2_ref, b2_ref, w3_ref, b3_ref,
         out_ref) = refs
    else:
        (xyzp_ref, xyzT_ref, nxyz_ref,
         w1x_ref, b1_ref, w2_ref, b2_ref, w3_ref, b3_ref, out_ref) = refs
    c = nxyz_ref[0]
    pT = xyzT_ref[0]
    d = -2.0 * jnp.dot(c, pT, preferred_element_type=jnp.float32)
    d = d + jnp.sum(c * c, axis=1, keepdims=True)
    d = d + jnp.sum(pT * pT, axis=0, keepdims=True)
    maskf = jnp.where(d > r2, 0.0, 1.0)
    rank = maskf
    k = 1
    while k < n:
        shifted = jnp.pad(rank[:, :n - k], ((0, 0), (k, 0)))
        rank = rank + shifted
        k *= 2
    count = rank[:, n - 1:n]
    jv = (1 + jax.lax.broadcasted_iota(jnp.int32, (1, ns, 1), 1)
          ).astype(jnp.float32)
    S = jnp.where(rank[:, None, :] == jv, maskf[:, None, :], 0.0)
    S2 = S.reshape(sc * ns, n)
    g = jnp.dot(S2, xyzp_ref[0], preferred_element_type=jnp.float32)
    cond = jv <= count[:, None, :]
    g3 = g.reshape(sc, ns, 8)
    g3 = jnp.where(cond, g3, g3[:, 0:1, :])
    x = (g3 - c[:, None, :]).reshape(sc * ns, 8)
    h = jnp.dot(x, w1x_ref[...], preferred_element_type=jnp.float32)
    if has_feats:
        gf = jnp.dot(S2, feats_ref[0], preferred_element_type=jnp.float32)
        gf3 = gf.reshape(sc, ns, gf.shape[-1])
        gf3 = jnp.where(cond, gf3, gf3[:, 0:1, :])
        gf2 = gf3.reshape(sc * ns, gf.shape[-1])
        h = h + jnp.dot(gf2, w1f_ref[...], preferred_element_type=jnp.float32)
    h = jax.nn.relu(h + b1_ref[...])
    h = jax.nn.relu(jnp.dot(h, w2_ref[...], preferred_element_type=jnp.float32) + b2_ref[...])
    h = jax.nn.relu(jnp.dot(h, w3_ref[...], preferred_element_type=jnp.float32) + b3_ref[...])
    out_ref[0] = jnp.max(h.reshape(sc, ns, h.shape[-1]), axis=1)


def _set_abstraction(xyzp, xyzT, feats, nxyz, layers, sc, ns, radius):
    n = xyzp.shape[1]
    s_tot = nxyz.shape[1]
    w1, b1 = _fold_bn(layers[0])
    w2, b2 = _fold_bn(layers[1])
    w3, b3 = _fold_bn(layers[2])
    w1x = _pad_rows(w1[:3], 8)
    c1, c2, c3 = w1.shape[1], w2.shape[1], w3.shape[1]
    has_feats = feats is not None
    full = lambda shp: pl.BlockSpec(shp, lambda b, s: (0,) * len(shp))
    in_specs = [pl.BlockSpec((1, n, 8), lambda b, s: (b, 0, 0)),
                pl.BlockSpec((1, 8, n), lambda b, s: (b, 0, 0))]
    args = [xyzp, xyzT]
    if has_feats:
        cf = feats.shape[-1]
        in_specs.append(pl.BlockSpec((1, n, cf), lambda b, s: (b, 0, 0)))
        args.append(feats)
    in_specs.append(pl.BlockSpec((1, sc, 8), lambda b, s: (b, s, 0)))
    args.append(nxyz)
    wlist = [w1x] + ([w1[3:]] if has_feats else []) + [b1[None], w2, b2[None], w3, b3[None]]
    for w in wlist:
        in_specs.append(full(w.shape))
        args.append(w)
    return pl.pallas_call(
        functools.partial(_sa_body, n, sc, ns, radius * radius, has_feats),
        grid=(_B, s_tot // sc),
        in_specs=in_specs,
        out_specs=pl.BlockSpec((1, sc, c3), lambda b, s: (b, s, 0)),
        out_shape=jax.ShapeDtypeStruct((_B, s_tot, c3), jnp.float32),
    )(*args)


def _sa3fp3_body(xyzp_ref, pts_ref, wa_ref, wb_ref, b1_ref, w2_ref, b2_ref,
                 w3_ref, b3_ref, wp_ref, wi_ref, fb1_ref, fw2_ref, fb2_ref,
                 out_ref):
    x = xyzp_ref[0]
    f = pts_ref[0]
    dot = lambda a, b: jnp.dot(a, b, preferred_element_type=jnp.float32)
    h = jax.nn.relu(dot(x, wa_ref[...]) + dot(f, wb_ref[...]) + b1_ref[...])
    h = jax.nn.relu(dot(h, w2_ref[...]) + b2_ref[...])
    h = jax.nn.relu(dot(h, w3_ref[...]) + b3_ref[...])
    l3 = jnp.max(h, axis=0, keepdims=True)
    g = jax.nn.relu(dot(f, wp_ref[...]) + dot(l3, wi_ref[...]) + fb1_ref[...])
    out_ref[0] = jax.nn.relu(dot(g, fw2_ref[...]) + fb2_ref[...])


def _nn3_weights(d, n):
    lane = jax.lax.broadcasted_iota(jnp.int32, d.shape, 1)
    ohs, ms = [], []
    for _ in range(3):
        m = jnp.min(d, axis=1, keepdims=True)
        i = jnp.min(jnp.where(d == m, lane, n), axis=1, keepdims=True)
        oh = (lane == i).astype(jnp.float32)
        d = jnp.where(oh > 0.0, 1e30, d)
        ohs.append(oh)
        ms.append(m)
    recips = [1.0 / (m + 1e-8) for m in ms]
    tot = recips[0] + recips[1] + recips[2]
    M = ohs[0] * (recips[0] / tot)
    M = M + ohs[1] * (recips[1] / tot)
    M = M + ohs[2] * (recips[2] / tot)
    return M


def _sqdist(x1, x2T):
    d = -2.0 * jnp.dot(x1, x2T, preferred_element_type=jnp.float32)
    d = d + jnp.sum(x1 * x1, axis=1, keepdims=True)
    return d + jnp.sum(x2T * x2T, axis=0, keepdims=True)


def _fp2_body(x1_ref, x2T_ref, pts1_ref, pts2_ref, wp_ref, wi_ref, b1_ref,
              w2_ref, b2_ref, out_ref):
    d = _sqdist(x1_ref[0], x2T_ref[0])
    M = _nn3_weights(d, d.shape[1])
    dot = lambda a, b: jnp.dot(a, b, preferred_element_type=jnp.float32)
    interp = dot(M, pts2_ref[0])
    h = jax.nn.relu(dot(pts1_ref[0], wp_ref[...]) + dot(interp, wi_ref[...])
                    + b1_ref[...])
    out_ref[0] = jax.nn.relu(dot(h, w2_ref[...]) + b2_ref[...])


def _fp1_body(x1_ref, x2T_ref, pts2_ref, w1_ref, b1_ref, w2_ref, b2_ref,
              w3_ref, b3_ref, wh_ref, bh_ref, wo_ref, bo_ref, out_ref):
    d = _sqdist(x1_ref[0], x2T_ref[0])
    M = _nn3_weights(d, d.shape[1])
    dot = lambda a, b: jnp.dot(a, b, preferred_element_type=jnp.float32)
    h = dot(M, pts2_ref[0])
    h = jax.nn.relu(dot(h, w1_ref[...]) + b1_ref[...])
    h = jax.nn.relu(dot(h, w2_ref[...]) + b2_ref[...])
    h = jax.nn.relu(dot(h, w3_ref[...]) + b3_ref[...])
    h = jax.nn.relu(dot(h, wh_ref[...]) + bh_ref[...])
    logits = dot(h, wo_ref[...]) + bo_ref[...]
    z = logits - jnp.max(logits, axis=1, keepdims=True)
    out_ref[0] = z - jnp.log(jnp.sum(jnp.exp(z), axis=1, keepdims=True))


def _full(shp):
    return pl.BlockSpec(shp, lambda *a: (0,) * len(shp))


def _batched(shp):
    nz = len(shp) - 1
    return pl.BlockSpec((1,) + shp[1:], lambda b, *a: (b,) + (0,) * nz)


def kernel(xyz, params):
    p = params
    xyzT = jnp.pad(xyz, ((0, 0), (0, 5), (0, 0)))
    xyzp = jnp.transpose(xyzT, (0, 2, 1))

    l1_xyz = _fps(xyzT, 512)
    l1_pts = _set_abstraction(xyzp, xyzT, None, l1_xyz, p['sa1'],
                              sc=32, ns=32, radius=0.2)
    l1_xyzT = jnp.transpose(l1_xyz, (0, 2, 1))
    l2_xyz = _fps(l1_xyzT, 128)
    l2_pts = _set_abstraction(l1_xyz, l1_xyzT, l1_pts, l2_xyz, p['sa2'],
                              sc=32, ns=64, radius=0.4)
    l2_xyzT = jnp.transpose(l2_xyz, (0, 2, 1))

    wa1, ab1 = _fold_bn(p['sa3'][0])
    wa2, ab2 = _fold_bn(p['sa3'][1])
    wa3, ab3 = _fold_bn(p['sa3'][2])
    wf1, fb1 = _fold_bn(p['fp3'][0])
    wf2, fb2 = _fold_bn(p['fp3'][1])
    ws = [_pad_rows(wa1[:3], 8), wa1[3:], ab1[None], wa2, ab2[None],
          wa3, ab3[None], wf1[:256], wf1[256:], fb1[None], wf2, fb2[None]]
    l2_pn = pl.pallas_call(
        _sa3fp3_body, grid=(_B,),
        in_specs=[_batched((_B, 128, 8)), _batched((_B, 128, 256))]
        + [_full(w.shape) for w in ws],
        out_specs=_batched((_B, 128, 256)),
        out_shape=jax.ShapeDtypeStruct((_B, 128, 256), jnp.float32),
    )(l2_xyz, l2_pts, *ws)

    wp1, pb1 = _fold_bn(p['fp2'][0])
    wp2, pb2 = _fold_bn(p['fp2'][1])
    ws = [wp1[:128], wp1[128:], pb1[None], wp2, pb2[None]]
    l1_pn = pl.pallas_call(
        _fp2_body, grid=(_B,),
        in_specs=[_batched((_B, 512, 8)), _batched((_B, 8, 128)),
                  _batched((_B, 512, 128)), _batched((_B, 128, 256))]
        + [_full(w.shape) for w in ws],
        out_specs=_batched((_B, 512, 128)),
        out_shape=jax.ShapeDtypeStruct((_B, 512, 128), jnp.float32),
    )(l1_xyz, l2_xyzT, l1_pts, l2_pn, *ws)

    wq1, qb1 = _fold_bn(p['fp1'][0])
    wq2, qb2 = _fold_bn(p['fp1'][1])
    wq3, qb3 = _fold_bn(p['fp1'][2])
    wh, hb = _fold_bn(p['head1'][0])
    wo = p['head2']['W'].T
    bo = p['head2']['b'][None]
    ws = [wq1, qb1[None], wq2, qb2[None], wq3, qb3[None], wh, hb[None], wo, bo]
    nc = 512
    out = pl.pallas_call(
        _fp1_body, grid=(_B, _N // nc),
        in_specs=[pl.BlockSpec((1, nc, 8), lambda b, s: (b, s, 0)),
                  _batched((_B, 8, 512)), _batched((_B, 512, 128))]
        + [_full(w.shape) for w in ws],
        out_specs=pl.BlockSpec((1, nc, _NPART), lambda b, s: (b, s, 0)),
        out_shape=jax.ShapeDtypeStruct((_B, _N, _NPART), jnp.float32),
    )(xyzp, l1_xyzT, l1_pn, *ws)
    return out


# speedup vs baseline: 23.9353x; 1.3816x over previous
import functools
import jax
import jax.numpy as jnp
from jax.experimental import pallas as pl

_B, _N, _NPART = 32, 2048, 50


def _fold_bn(layer):
    s = layer['gamma'] / jnp.sqrt(layer['var'] + 1e-5)
    wt = (layer['W'] * s[:, None]).T
    b = (layer['b'] - layer['mean']) * s + layer['beta']
    return wt, b


def _pad_rows(wt, to):
    return jnp.pad(wt, ((0, to - wt.shape[0]), (0, 0)))


def _fps_body(n, npoint, x_ref, y_ref, z_ref, out_ref):
    x, y, z = x_ref[...], y_ref[...], z_ref[...]
    lane = jax.lax.broadcasted_iota(jnp.int32, (_B, n), 1)
    zero5 = jnp.zeros((_B, 5), dtype=jnp.float32)

    def body(i, state):
        dist, far = state
        onehot = (lane == far).astype(jnp.float32)
        cx = jnp.sum(x * onehot, axis=1, keepdims=True)
        cy = jnp.sum(y * onehot, axis=1, keepdims=True)
        cz = jnp.sum(z * onehot, axis=1, keepdims=True)
        c = jnp.concatenate([cx, cy, cz, zero5], axis=1)
        out_ref[:, pl.ds(i, 1), :] = c[:, None, :]
        d = (x - cx) ** 2 + (y - cy) ** 2 + (z - cz) ** 2
        dist = jnp.minimum(dist, d)
        m = jnp.max(dist, axis=1, keepdims=True)
        far = jnp.min(jnp.where(dist == m, lane, n),
                      axis=1, keepdims=True).astype(jnp.int32)
        return dist, far

    dist0 = jnp.full((_B, n), 1e10, dtype=jnp.float32)
    far0 = jnp.zeros((_B, 1), dtype=jnp.int32)
    jax.lax.fori_loop(0, npoint, body, (dist0, far0))


def _fps(x, y, z, npoint):
    n = x.shape[1]
    return pl.pallas_call(
        functools.partial(_fps_body, n, npoint),
        out_shape=jax.ShapeDtypeStruct((_B, npoint, 8), jnp.float32),
    )(x, y, z)


def _sa_body(n, sc, ns, r2, has_feats, *refs):
    if has_feats:
        (xyzp_ref, xyzT_ref, feats_ref, nxyz_ref,
         w1x_ref, w1f_ref, b1_ref, w2_ref, b2_ref, w3_ref, b3_ref,
         out_ref) = refs
    else:
        (xyzp_ref, xyzT_ref, nxyz_ref,
         w1x_ref, b1_ref, w2_ref, b2_ref, w3_ref, b3_ref, out_ref) = refs
    c = nxyz_ref[0]
    pT = xyzT_ref[0]
    d = -2.0 * jnp.dot(c, pT, preferred_element_type=jnp.float32)
    d = d + jnp.sum(c * c, axis=1, keepdims=True)
    d = d + jnp.sum(pT * pT, axis=0, keepdims=True)
    maskf = jnp.where(d > r2, 0.0, 1.0)
    rank = maskf
    k = 1
    while k < n:
        shifted = jnp.pad(rank[:, :n - k], ((0, 0), (k, 0)))
        rank = rank + shifted
        k *= 2
    count = rank[:, n - 1:n]
    jv = (1 + jax.lax.broadcasted_iota(jnp.int32, (1, ns, 1), 1)
          ).astype(jnp.float32)
    rm = rank * maskf
    S = (rm[:, None, :] == jv).astype(jnp.float32)
    S2 = S.reshape(sc * ns, n)
    g = jnp.dot(S2, xyzp_ref[0], preferred_element_type=jnp.float32)
    cond = jv <= count[:, None, :]
    g3 = g.reshape(sc, ns, 8)
    g3 = jnp.where(cond, g3, g3[:, 0:1, :])
    x = (g3 - c[:, None, :]).reshape(sc * ns, 8)
    h = jnp.dot(x, w1x_ref[...], preferred_element_type=jnp.float32)
    if has_feats:
        gf = jnp.dot(S2, feats_ref[0], preferred_element_type=jnp.float32)
        gf3 = gf.reshape(sc, ns, gf.shape[-1])
        gf3 = jnp.where(cond, gf3, gf3[:, 0:1, :])
        gf2 = gf3.reshape(sc * ns, gf.shape[-1])
        h = h + jnp.dot(gf2, w1f_ref[...], preferred_element_type=jnp.float32)
    h = jax.nn.relu(h + b1_ref[...])
    h = jax.nn.relu(jnp.dot(h, w2_ref[...], preferred_element_type=jnp.float32) + b2_ref[...])
    h = jax.nn.relu(jnp.dot(h, w3_ref[...], preferred_element_type=jnp.float32) + b3_ref[...])
    out_ref[0] = jnp.max(h.reshape(sc, ns, h.shape[-1]), axis=1)


def _set_abstraction(xyzp, xyzT, feats, nxyz, layers, sc, ns, radius):
    n = xyzp.shape[1]
    s_tot = nxyz.shape[1]
    w1, b1 = _fold_bn(layers[0])
    w2, b2 = _fold_bn(layers[1])
    w3, b3 = _fold_bn(layers[2])
    w1x = _pad_rows(w1[:3], 8)
    c1, c2, c3 = w1.shape[1], w2.shape[1], w3.shape[1]
    has_feats = feats is not None
    full = lambda shp: pl.BlockSpec(shp, lambda b, s: (0,) * len(shp))
    in_specs = [pl.BlockSpec((1, n, 8), lambda b, s: (b, 0, 0)),
                pl.BlockSpec((1, 8, n), lambda b, s: (b, 0, 0))]
    args = [xyzp, xyzT]
    if has_feats:
        cf = feats.shape[-1]
        in_specs.append(pl.BlockSpec((1, n, cf), lambda b, s: (b, 0, 0)))
        args.append(feats)
    in_specs.append(pl.BlockSpec((1, sc, 8), lambda b, s: (b, s, 0)))
    args.append(nxyz)
    wlist = [w1x] + ([w1[3:]] if has_feats else []) + [b1[None], w2, b2[None], w3, b3[None]]
    for w in wlist:
        in_specs.append(full(w.shape))
        args.append(w)
    return pl.pallas_call(
        functools.partial(_sa_body, n, sc, ns, radius * radius, has_feats),
        grid=(_B, s_tot // sc),
        in_specs=in_specs,
        out_specs=pl.BlockSpec((1, sc, c3), lambda b, s: (b, s, 0)),
        out_shape=jax.ShapeDtypeStruct((_B, s_tot, c3), jnp.float32),
    )(*args)


def _sa3fp3_body(xyzp_ref, pts_ref, wa_ref, wb_ref, b1_ref, w2_ref, b2_ref,
                 w3_ref, b3_ref, wp_ref, wi_ref, fb1_ref, fw2_ref, fb2_ref,
                 out_ref):
    x = xyzp_ref[0]
    f = pts_ref[0]
    dot = lambda a, b: jnp.dot(a, b, preferred_element_type=jnp.float32)
    h = jax.nn.relu(dot(x, wa_ref[...]) + dot(f, wb_ref[...]) + b1_ref[...])
    h = jax.nn.relu(dot(h, w2_ref[...]) + b2_ref[...])
    h = jax.nn.relu(dot(h, w3_ref[...]) + b3_ref[...])
    l3 = jnp.max(h, axis=0, keepdims=True)
    g = jax.nn.relu(dot(f, wp_ref[...]) + dot(l3, wi_ref[...]) + fb1_ref[...])
    out_ref[0] = jax.nn.relu(dot(g, fw2_ref[...]) + fb2_ref[...])


def _nn3_weights(d, n):
    lane = jax.lax.broadcasted_iota(jnp.int32, d.shape, 1)
    ohs, ms = [], []
    for _ in range(3):
        m = jnp.min(d, axis=1, keepdims=True)
        i = jnp.min(jnp.where(d == m, lane, n), axis=1, keepdims=True)
        oh = (lane == i).astype(jnp.float32)
        d = jnp.where(oh > 0.0, 1e30, d)
        ohs.append(oh)
        ms.append(m)
    recips = [1.0 / (m + 1e-8) for m in ms]
    tot = recips[0] + recips[1] + recips[2]
    M = ohs[0] * (recips[0] / tot)
    M = M + ohs[1] * (recips[1] / tot)
    M = M + ohs[2] * (recips[2] / tot)
    return M


def _sqdist(x1, x2T):
    d = -2.0 * jnp.dot(x1, x2T, preferred_element_type=jnp.float32)
    d = d + jnp.sum(x1 * x1, axis=1, keepdims=True)
    return d + jnp.sum(x2T * x2T, axis=0, keepdims=True)


def _fp2_body(x1_ref, x2T_ref, pts1_ref, pts2_ref, wp_ref, wi_ref, b1_ref,
              w2_ref, b2_ref, out_ref):
    d = _sqdist(x1_ref[0], x2T_ref[0])
    M = _nn3_weights(d, d.shape[1])
    dot = lambda a, b: jnp.dot(a, b, preferred_element_type=jnp.float32)
    interp = dot(M, pts2_ref[0])
    h = jax.nn.relu(dot(pts1_ref[0], wp_ref[...]) + dot(interp, wi_ref[...])
                    + b1_ref[...])
    out_ref[0] = jax.nn.relu(dot(h, w2_ref[...]) + b2_ref[...])


def _fp1_body(x1_ref, x2T_ref, pts2_ref, w1_ref, b1_ref, w2_ref, b2_ref,
              w3_ref, b3_ref, wh_ref, bh_ref, wo_ref, bo_ref, out_ref):
    d = _sqdist(x1_ref[0], x2T_ref[0])
    M = _nn3_weights(d, d.shape[1])
    dot = lambda a, b: jnp.dot(a, b, preferred_element_type=jnp.float32)
    h = dot(M, pts2_ref[0])
    h = jax.nn.relu(dot(h, w1_ref[...]) + b1_ref[...])
    h = jax.nn.relu(dot(h, w2_ref[...]) + b2_ref[...])
    h = jax.nn.relu(dot(h, w3_ref[...]) + b3_ref[...])
    h = jax.nn.relu(dot(h, wh_ref[...]) + bh_ref[...])
    logits = dot(h, wo_ref[...]) + bo_ref[...]
    z = logits - jnp.max(logits, axis=1, keepdims=True)
    out_ref[0] = z - jnp.log(jnp.sum(jnp.exp(z), axis=1, keepdims=True))


def _full(shp):
    return pl.BlockSpec(shp, lambda *a: (0,) * len(shp))


def _batched(shp):
    nz = len(shp) - 1
    return pl.BlockSpec((1,) + shp[1:], lambda b, *a: (b,) + (0,) * nz)


def kernel(xyz, params):
    p = params
    xyzT = jnp.pad(xyz, ((0, 0), (0, 5), (0, 0)))
    xyzp = jnp.transpose(xyzT, (0, 2, 1))

    l1_xyz = _fps(xyz[:, 0], xyz[:, 1], xyz[:, 2], 512)
    l1_pts = _set_abstraction(xyzp, xyzT, None, l1_xyz, p['sa1'],
                              sc=64, ns=32, radius=0.2)
    l1_xyzT = jnp.transpose(l1_xyz, (0, 2, 1))
    l2_xyz = _fps(l1_xyz[:, :, 0], l1_xyz[:, :, 1],
                  l1_xyz[:, :, 2], 128)
    l2_pts = _set_abstraction(l1_xyz, l1_xyzT, l1_pts, l2_xyz, p['sa2'],
                              sc=64, ns=64, radius=0.4)
    l2_xyzT = jnp.transpose(l2_xyz, (0, 2, 1))

    wa1, ab1 = _fold_bn(p['sa3'][0])
    wa2, ab2 = _fold_bn(p['sa3'][1])
    wa3, ab3 = _fold_bn(p['sa3'][2])
    wf1, fb1 = _fold_bn(p['fp3'][0])
    wf2, fb2 = _fold_bn(p['fp3'][1])
    ws = [_pad_rows(wa1[:3], 8), wa1[3:], ab1[None], wa2, ab2[None],
          wa3, ab3[None], wf1[:256], wf1[256:], fb1[None], wf2, fb2[None]]
    l2_pn = pl.pallas_call(
        _sa3fp3_body, grid=(_B,),
        in_specs=[_batched((_B, 128, 8)), _batched((_B, 128, 256))]
        + [_full(w.shape) for w in ws],
        out_specs=_batched((_B, 128, 256)),
        out_shape=jax.ShapeDtypeStruct((_B, 128, 256), jnp.float32),
    )(l2_xyz, l2_pts, *ws)

    wp1, pb1 = _fold_bn(p['fp2'][0])
    wp2, pb2 = _fold_bn(p['fp2'][1])
    ws = [wp1[:128], wp1[128:], pb1[None], wp2, pb2[None]]
    l1_pn = pl.pallas_call(
        _fp2_body, grid=(_B,),
        in_specs=[_batched((_B, 512, 8)), _batched((_B, 8, 128)),
                  _batched((_B, 512, 128)), _batched((_B, 128, 256))]
        + [_full(w.shape) for w in ws],
        out_specs=_batched((_B, 512, 128)),
        out_shape=jax.ShapeDtypeStruct((_B, 512, 128), jnp.float32),
    )(l1_xyz, l2_xyzT, l1_pts, l2_pn, *ws)

    wq1, qb1 = _fold_bn(p['fp1'][0])
    wq2, qb2 = _fold_bn(p['fp1'][1])
    wq3, qb3 = _fold_bn(p['fp1'][2])
    wh, hb = _fold_bn(p['head1'][0])
    wo = p['head2']['W'].T
    bo = p['head2']['b'][None]
    ws = [wq1, qb1[None], wq2, qb2[None], wq3, qb3[None], wh, hb[None], wo, bo]
    nc = 512
    out = pl.pallas_call(
        _fp1_body, grid=(_B, _N // nc),
        in_specs=[pl.BlockSpec((1, nc, 8), lambda b, s: (b, s, 0)),
                  _batched((_B, 8, 512)), _batched((_B, 512, 128))]
        + [_full(w.shape) for w in ws],
        out_specs=pl.BlockSpec((1, nc, _NPART), lambda b, s: (b, s, 0)),
        out_shape=jax.ShapeDtypeStruct((_B, _N, _NPART), jnp.float32),
    )(xyzp, l1_xyzT, l1_pn, *ws)
    return out


# speedup vs baseline: 25.9497x; 1.0842x over previous
import functools
import jax
import jax.numpy as jnp
from jax.experimental import pallas as pl

_B, _N, _NPART = 32, 2048, 50


def _fold_bn(layer):
    s = layer['gamma'] / jnp.sqrt(layer['var'] + 1e-5)
    wt = (layer['W'] * s[:, None]).T
    b = (layer['b'] - layer['mean']) * s + layer['beta']
    return wt, b


def _pad_rows(wt, to):
    return jnp.pad(wt, ((0, to - wt.shape[0]), (0, 0)))


def _fps_body(n, npoint, x_ref, y_ref, z_ref, out_ref):
    x, y, z = x_ref[...], y_ref[...], z_ref[...]
    lane = jax.lax.broadcasted_iota(jnp.int32, (_B, n), 1)
    zero5 = jnp.zeros((_B, 5), dtype=jnp.float32)

    def body(i, state):
        dist, far = state
        onehot = (lane == far).astype(jnp.float32)
        cx = jnp.sum(x * onehot, axis=1, keepdims=True)
        cy = jnp.sum(y * onehot, axis=1, keepdims=True)
        cz = jnp.sum(z * onehot, axis=1, keepdims=True)
        c = jnp.concatenate([cx, cy, cz, zero5], axis=1)
        out_ref[:, pl.ds(i, 1), :] = c[:, None, :]
        d = (x - cx) ** 2 + (y - cy) ** 2 + (z - cz) ** 2
        dist = jnp.minimum(dist, d)
        m = jnp.max(dist, axis=1, keepdims=True)
        far = jnp.min(jnp.where(dist == m, lane, n),
                      axis=1, keepdims=True).astype(jnp.int32)
        return dist, far

    dist0 = jnp.full((_B, n), 1e10, dtype=jnp.float32)
    far0 = jnp.zeros((_B, 1), dtype=jnp.int32)
    jax.lax.fori_loop(0, npoint, body, (dist0, far0))


def _fps(x, y, z, npoint):
    n = x.shape[1]
    return pl.pallas_call(
        functools.partial(_fps_body, n, npoint),
        out_shape=jax.ShapeDtypeStruct((_B, npoint, 8), jnp.float32),
    )(x, y, z)


def _sa_body(n, sc, ns, r2, has_feats, *refs):
    if has_feats:
        (xyzp_ref, xyzT_ref, feats_ref, nxyz_ref,
         w1x_ref, w1f_ref, b1_ref, w2_ref, b2_ref, w3_ref, b3_ref,
         out_ref) = refs
    else:
        (xyzp_ref, xyzT_ref, nxyz_ref,
         w1x_ref, b1_ref, w2_ref, b2_ref, w3_ref, b3_ref, out_ref) = refs
    c = nxyz_ref[0]
    pT = xyzT_ref[0]
    d = -2.0 * jnp.dot(c, pT, preferred_element_type=jnp.float32)
    d = d + jnp.sum(c * c, axis=1, keepdims=True)
    d = d + jnp.sum(pT * pT, axis=0, keepdims=True)
    maskf = jnp.where(d > r2, 0.0, 1.0)
    rank = maskf
    k = 1
    while k < n:
        shifted = jnp.pad(rank[:, :n - k], ((0, 0), (k, 0)))
        rank = rank + shifted
        k *= 2
    count = rank[:, n - 1:n]
    jv = (1 + jax.lax.broadcasted_iota(jnp.int32, (1, ns, 1), 1)
          ).astype(jnp.float32)
    rm = rank * maskf
    S = (rm[:, None, :] == jv).astype(jnp.float32)
    S2 = S.reshape(sc * ns, n)
    G = jnp.dot(xyzp_ref[0], w1x_ref[...], preferred_element_type=jnp.float32)
    if has_feats:
        G = G + jnp.dot(feats_ref[0], w1f_ref[...],
                        preferred_element_type=jnp.float32)
    c1 = G.shape[-1]
    sg = jnp.dot(S2, G, preferred_element_type=jnp.float32).reshape(sc, ns, c1)
    cond = jv <= count[:, None, :]
    sg = jnp.where(cond, sg, sg[:, 0:1, :])
    cw = jnp.dot(c, w1x_ref[...], preferred_element_type=jnp.float32)
    h = jax.nn.relu(sg - cw[:, None, :] + b1_ref[...]).reshape(sc * ns, c1)
    h = jax.nn.relu(jnp.dot(h, w2_ref[...], preferred_element_type=jnp.float32) + b2_ref[...])
    h = jax.nn.relu(jnp.dot(h, w3_ref[...], preferred_element_type=jnp.float32) + b3_ref[...])
    out_ref[0] = jnp.max(h.reshape(sc, ns, h.shape[-1]), axis=1)


def _set_abstraction(xyzp, xyzT, feats, nxyz, layers, sc, ns, radius):
    n = xyzp.shape[1]
    s_tot = nxyz.shape[1]
    w1, b1 = _fold_bn(layers[0])
    w2, b2 = _fold_bn(layers[1])
    w3, b3 = _fold_bn(layers[2])
    w1x = _pad_rows(w1[:3], 8)
    c1, c2, c3 = w1.shape[1], w2.shape[1], w3.shape[1]
    has_feats = feats is not None
    full = lambda shp: pl.BlockSpec(shp, lambda b, s: (0,) * len(shp))
    in_specs = [pl.BlockSpec((1, n, 8), lambda b, s: (b, 0, 0)),
                pl.BlockSpec((1, 8, n), lambda b, s: (b, 0, 0))]
    args = [xyzp, xyzT]
    if has_feats:
        cf = feats.shape[-1]
        in_specs.append(pl.BlockSpec((1, n, cf), lambda b, s: (b, 0, 0)))
        args.append(feats)
    in_specs.append(pl.BlockSpec((1, sc, 8), lambda b, s: (b, s, 0)))
    args.append(nxyz)
    wlist = [w1x] + ([w1[3:]] if has_feats else []) + [b1[None], w2, b2[None], w3, b3[None]]
    for w in wlist:
        in_specs.append(full(w.shape))
        args.append(w)
    return pl.pallas_call(
        functools.partial(_sa_body, n, sc, ns, radius * radius, has_feats),
        grid=(_B, s_tot // sc),
        in_specs=in_specs,
        out_specs=pl.BlockSpec((1, sc, c3), lambda b, s: (b, s, 0)),
        out_shape=jax.ShapeDtypeStruct((_B, s_tot, c3), jnp.float32),
    )(*args)


def _sa3fp3_body(xyzp_ref, pts_ref, wa_ref, wb_ref, b1_ref, w2_ref, b2_ref,
                 w3_ref, b3_ref, wp_ref, wi_ref, fb1_ref, fw2_ref, fb2_ref,
                 out_ref):
    x = xyzp_ref[0]
    f = pts_ref[0]
    dot = lambda a, b: jnp.dot(a, b, preferred_element_type=jnp.float32)
    h = jax.nn.relu(dot(x, wa_ref[...]) + dot(f, wb_ref[...]) + b1_ref[...])
    h = jax.nn.relu(dot(h, w2_ref[...]) + b2_ref[...])
    h = jax.nn.relu(dot(h, w3_ref[...]) + b3_ref[...])
    l3 = jnp.max(h, axis=0, keepdims=True)
    g = jax.nn.relu(dot(f, wp_ref[...]) + dot(l3, wi_ref[...]) + fb1_ref[...])
    out_ref[0] = jax.nn.relu(dot(g, fw2_ref[...]) + fb2_ref[...])


def _nn3_weights(d, n):
    lane = jax.lax.broadcasted_iota(jnp.int32, d.shape, 1)
    ohs, ms = [], []
    for _ in range(3):
        m = jnp.min(d, axis=1, keepdims=True)
        i = jnp.min(jnp.where(d == m, lane, n), axis=1, keepdims=True)
        oh = (lane == i).astype(jnp.float32)
        d = jnp.where(oh > 0.0, 1e30, d)
        ohs.append(oh)
        ms.append(m)
    recips = [1.0 / (m + 1e-8) for m in ms]
    tot = recips[0] + recips[1] + recips[2]
    M = ohs[0] * (recips[0] / tot)
    M = M + ohs[1] * (recips[1] / tot)
    M = M + ohs[2] * (recips[2] / tot)
    return M


def _sqdist(x1, x2T):
    d = -2.0 * jnp.dot(x1, x2T, preferred_element_type=jnp.float32)
    d = d + jnp.sum(x1 * x1, axis=1, keepdims=True)
    return d + jnp.sum(x2T * x2T, axis=0, keepdims=True)


def _fp2_body(x1_ref, x2T_ref, pts1_ref, pts2_ref, wp_ref, wi_ref, b1_ref,
              w2_ref, b2_ref, out_ref):
    d = _sqdist(x1_ref[0], x2T_ref[0])
    M = _nn3_weights(d, d.shape[1])
    dot = lambda a, b: jnp.dot(a, b, preferred_element_type=jnp.float32)
    interp = dot(M, pts2_ref[0])
    h = jax.nn.relu(dot(pts1_ref[0], wp_ref[...]) + dot(interp, wi_ref[...])
                    + b1_ref[...])
    out_ref[0] = jax.nn.relu(dot(h, w2_ref[...]) + b2_ref[...])


def _fp1_body(x1_ref, x2T_ref, pts2_ref, w1_ref, b1_ref, w2_ref, b2_ref,
              w3_ref, b3_ref, wh_ref, bh_ref, wo_ref, bo_ref, out_ref):
    d = _sqdist(x1_ref[0], x2T_ref[0])
    M = _nn3_weights(d, d.shape[1])
    dot = lambda a, b: jnp.dot(a, b, preferred_element_type=jnp.float32)
    h = dot(M, pts2_ref[0])
    h = jax.nn.relu(dot(h, w1_ref[...]) + b1_ref[...])
    h = jax.nn.relu(dot(h, w2_ref[...]) + b2_ref[...])
    h = jax.nn.relu(dot(h, w3_ref[...]) + b3_ref[...])
    h = jax.nn.relu(dot(h, wh_ref[...]) + bh_ref[...])
    logits = dot(h, wo_ref[...]) + bo_ref[...]
    z = logits - jnp.max(logits, axis=1, keepdims=True)
    out_ref[0] = z - jnp.log(jnp.sum(jnp.exp(z), axis=1, keepdims=True))


def _full(shp):
    return pl.BlockSpec(shp, lambda *a: (0,) * len(shp))


def _batched(shp):
    nz = len(shp) - 1
    return pl.BlockSpec((1,) + shp[1:], lambda b, *a: (b,) + (0,) * nz)


def kernel(xyz, params):
    p = params
    xyzT = jnp.pad(xyz, ((0, 0), (0, 5), (0, 0)))
    xyzp = jnp.transpose(xyzT, (0, 2, 1))

    l1_xyz = _fps(xyz[:, 0], xyz[:, 1], xyz[:, 2], 512)
    l1_pts = _set_abstraction(xyzp, xyzT, None, l1_xyz, p['sa1'],
                              sc=64, ns=32, radius=0.2)
    l1_xyzT = jnp.transpose(l1_xyz, (0, 2, 1))
    l2_xyz = _fps(l1_xyz[:, :, 0], l1_xyz[:, :, 1],
                  l1_xyz[:, :, 2], 128)
    l2_pts = _set_abstraction(l1_xyz, l1_xyzT, l1_pts, l2_xyz, p['sa2'],
                              sc=64, ns=64, radius=0.4)
    l2_xyzT = jnp.transpose(l2_xyz, (0, 2, 1))

    wa1, ab1 = _fold_bn(p['sa3'][0])
    wa2, ab2 = _fold_bn(p['sa3'][1])
    wa3, ab3 = _fold_bn(p['sa3'][2])
    wf1, fb1 = _fold_bn(p['fp3'][0])
    wf2, fb2 = _fold_bn(p['fp3'][1])
    ws = [_pad_rows(wa1[:3], 8), wa1[3:], ab1[None], wa2, ab2[None],
          wa3, ab3[None], wf1[:256], wf1[256:], fb1[None], wf2, fb2[None]]
    l2_pn = pl.pallas_call(
        _sa3fp3_body, grid=(_B,),
        in_specs=[_batched((_B, 128, 8)), _batched((_B, 128, 256))]
        + [_full(w.shape) for w in ws],
        out_specs=_batched((_B, 128, 256)),
        out_shape=jax.ShapeDtypeStruct((_B, 128, 256), jnp.float32),
    )(l2_xyz, l2_pts, *ws)

    wp1, pb1 = _fold_bn(p['fp2'][0])
    wp2, pb2 = _fold_bn(p['fp2'][1])
    ws = [wp1[:128], wp1[128:], pb1[None], wp2, pb2[None]]
    l1_pn = pl.pallas_call(
        _fp2_body, grid=(_B,),
        in_specs=[_batched((_B, 512, 8)), _batched((_B, 8, 128)),
                  _batched((_B, 512, 128)), _batched((_B, 128, 256))]
        + [_full(w.shape) for w in ws],
        out_specs=_batched((_B, 512, 128)),
        out_shape=jax.ShapeDtypeStruct((_B, 512, 128), jnp.float32),
    )(l1_xyz, l2_xyzT, l1_pts, l2_pn, *ws)

    wq1, qb1 = _fold_bn(p['fp1'][0])
    wq2, qb2 = _fold_bn(p['fp1'][1])
    wq3, qb3 = _fold_bn(p['fp1'][2])
    wh, hb = _fold_bn(p['head1'][0])
    wo = p['head2']['W'].T
    bo = p['head2']['b'][None]
    ws = [wq1, qb1[None], wq2, qb2[None], wq3, qb3[None], wh, hb[None], wo, bo]
    nc = 512
    out = pl.pallas_call(
        _fp1_body, grid=(_B, _N // nc),
        in_specs=[pl.BlockSpec((1, nc, 8), lambda b, s: (b, s, 0)),
                  _batched((_B, 8, 512)), _batched((_B, 512, 128))]
        + [_full(w.shape) for w in ws],
        out_specs=pl.BlockSpec((1, nc, _NPART), lambda b, s: (b, s, 0)),
        out_shape=jax.ShapeDtypeStruct((_B, _N, _NPART), jnp.float32),
    )(xyzp, l1_xyzT, l1_pn, *ws)
    return out
